# Initial kernel scaffold; baseline (speedup 1.0000x reference)
import jax
import jax.numpy as jnp
from jax.experimental import pallas as pl


def kernel(x, conv_w, centroids):
    raise NotImplementedError("write your pallas kernel here")


# speedup vs baseline: 2.7289x; 2.7289x over previous
import jax
import jax.numpy as jnp
from jax.experimental import pallas as pl
from jax.experimental.pallas import tpu as pltpu

_EPS = 1e-12
_NCH = 16
_HI = jax.lax.Precision.HIGHEST


def _box4(a):
    p = a[:, 0:1199] + a[:, 1:1200]
    h = p[:, 0:1197] + p[:, 2:1199]
    q = h[:, 0:1157] + h[:, 40:1197]
    return q[:, 0:1077] + q[:, 80:1157]


def _body(x_ref, w_ref, cf_ref, c_ref, g_ref, l_ref,
          xn_s, sa_s, sv_s, y_s, tsq_s):
    t = pl.program_id(1)

    @pl.when(t == 0)
    def _setup():
        xm = x_ref[0]
        nrm = jnp.sqrt(jnp.sum(xm * xm, axis=0, keepdims=True))
        xn = xm / jnp.maximum(nrm, _EPS)
        xn_s[...] = xn
        logits = jax.lax.dot_general(
            w_ref[...], xn, (((1,), (0,)), ((), ())),
            preferred_element_type=jnp.float32, precision=_HI)
        mx = jnp.max(logits, axis=0, keepdims=True)
        e = jnp.exp(logits - mx)
        sa = e / jnp.sum(e, axis=0, keepdims=True)
        sa_s[...] = sa.reshape(_NCH, 4, 1200)
        sv_s[...] = _box4(sa).reshape(_NCH, 4, 1077)
        m = jax.lax.dot_general(
            sa, xn, (((1,), (1,)), ((), ())),
            preferred_element_type=jnp.float32, precision=_HI)
        ssum = jnp.sum(sa, axis=1, keepdims=True)
        g = m - cf_ref[...] * ssum
        gn = g / jnp.maximum(
            jnp.sqrt(jnp.sum(g * g, axis=1, keepdims=True)), _EPS)
        tot = jnp.sqrt(jnp.sum(gn * gn))
        g_ref[...] = (gn / jnp.maximum(tot, _EPS)).reshape(1, 1, 8192)

    @pl.when(t < _NCH)
    def _compute():
        sa4 = sa_s[t]
        xn = xn_s[...]
        z = (sa4[:, None, :] * xn[None, :, :]).reshape(512, 1200)
        v = _box4(z).reshape(4, 128, 1077)
        c4 = c_ref[0]
        s4 = sv_s[t]
        y = (v - c4[:, :, None] * s4[:, None, :]) * jnp.float32(0.0625)
        nn = jnp.sqrt(jnp.sum(y * y, axis=1, keepdims=True))
        y = (y / jnp.maximum(nn, _EPS)).reshape(512, 1077)
        y_s[t] = y
        contrib = jnp.sum(y * y, axis=0, keepdims=True)
        prev = jnp.where(t == 0, jnp.zeros_like(contrib), tsq_s[...])
        tsq_s[...] = prev + contrib

    @pl.when(t >= _NCH)
    def _scale():
        j = t - _NCH
        inv = 1.0 / jnp.maximum(jnp.sqrt(tsq_s[...]), _EPS)
        v = y_s[j] * inv
        vp = jnp.concatenate([v, jnp.zeros((512, 3), jnp.float32)], axis=1)
        out = vp.reshape(512, 27, 40)[:, :, :37].reshape(512, 999)
        l_ref[...] = out.reshape(1, 512, 999)


def kernel(x, conv_w, centroids):
    xr = x.reshape(4, 128, 1200)
    cr = centroids.reshape(_NCH, 4, 128)
    vg, vl = pl.pallas_call(
        _body,
        grid=(4, 2 * _NCH),
        in_specs=[
            pl.BlockSpec((1, 128, 1200), lambda n, t: (n, 0, 0)),
            pl.BlockSpec((64, 128), lambda n, t: (0, 0)),
            pl.BlockSpec((64, 128), lambda n, t: (0, 0)),
            pl.BlockSpec((1, 4, 128),
                         lambda n, t: (jnp.minimum(t, _NCH - 1), 0, 0)),
        ],
        out_specs=[
            pl.BlockSpec((1, 1, 8192), lambda n, t: (n, 0, 0)),
            pl.BlockSpec((1, 512, 999),
                         lambda n, t: (n, jnp.maximum(t - _NCH, 0), 0)),
        ],
        out_shape=[
            jax.ShapeDtypeStruct((4, 1, 8192), jnp.float32),
            jax.ShapeDtypeStruct((4, 8192, 999), jnp.float32),
        ],
        scratch_shapes=[
            pltpu.VMEM((128, 1200), jnp.float32),
            pltpu.VMEM((_NCH, 4, 1200), jnp.float32),
            pltpu.VMEM((_NCH, 4, 1077), jnp.float32),
            pltpu.VMEM((_NCH, 512, 1077), jnp.float32),
            pltpu.VMEM((1, 1077), jnp.float32),
        ],
        compiler_params=pltpu.CompilerParams(
            dimension_semantics=("parallel", "arbitrary"),
        ),
    )(xr, conv_w, centroids, cr)
    return vg.reshape(4, 8192), vl
